# baseline (speedup 1.0000x reference)
import jax
import jax.numpy as jnp
import numpy as np
from jax.experimental import pallas as pl
from jax.experimental.pallas import tpu as pltpu

_B = 64
_M = 128
_F = 48
_C = 7
_H = 20
_N = _B * _M

_GP = 16
_GM = _GP * _M

_SQRT2 = 1.4142135623730951

_REAL = np.array([1.0] * _C + [0.0], dtype=np.float32)
_G_BLOCK = np.kron(np.eye(_C, dtype=np.float32),
                   np.outer(np.ones(8, dtype=np.float32), _REAL))
_SEL = np.kron(np.eye(_GP, dtype=np.float32), np.ones((1, _M), np.float32))

_RW_M1A, _RW_WV1S, _RW_M2A, _RW_M2S = 0, 128, 320, 368
_RW_L1, _RW_L2, _RW_P8, _WROWS = 416, 464, 480, 496
_RB_1A, _RB_1S, _RB_2A, _RB_2S = 0, 64, 160, 184
_RB_L1, _RB_L2, _RB_F, _BROWS = 208, 248, 256, 264


def _gelu(x):
    return 0.5 * x * (1.0 + jax.lax.erf(x / _SQRT2))


def _elu(x):
    return jnp.maximum(x, 0.0) + (jnp.exp(jnp.minimum(x, 0.0)) - 1.0)


def _b16(x):
    return x.astype(jnp.bfloat16)


def _mm(a, b):
    return jnp.dot(a, b, preferred_element_type=jnp.float32)


def _dotT(Wt, xb):
    return jax.lax.dot_general(Wt, xb, (((1,), (1,)), ((), ())),
                               preferred_element_type=jnp.float32)


def _fused_kernel(Aa_ref, Aw_ref, Ae_ref, Al_ref,
                  xT_ref, xTw_ref, xTe_ref, xD_ref, xDu_ref, xL_ref,
                  Wb_ref, Bb_ref, G8_ref, Sel_ref,
                  out_ref):
    sel = Sel_ref[...]

    def prep(Aref):
        Ab = _b16(Aref[...]).reshape(_GM, _M)
        deg = _mm(sel, Ab)
        safe = jnp.where(deg > 0, deg, 1.0)
        d = jnp.where(deg > 0, jax.lax.rsqrt(safe), 0.0)
        return Ab, d

    Aa, da = prep(Aa_ref)
    Aw, dw = prep(Aw_ref)
    Ae, de = prep(Ae_ref)
    Al, dl = prep(Al_ref)

    def agg(Ab, d, hT):
        parts = []
        for i in range(_GP):
            di = d[i:i + 1]
            p = _mm(_b16(hT[:, i * _M:(i + 1) * _M] * di),
                    Ab[i * _M:(i + 1) * _M])
            parts.append(p * di)
        return jnp.concatenate(parts, axis=1)

    def rs(ref):
        return _b16(ref[...]).reshape(_GM, _F)

    xa = jnp.concatenate([rs(xT_ref), rs(xD_ref), rs(xDu_ref)], axis=1)
    xLb = rs(xL_ref)
    H1a = _dotT(Wb_ref[_RW_M1A:_RW_M1A + 128], xa)
    h1a = _elu(_gelu(agg(Aa, da, H1a[0:64]) + H1a[64:128]
                     + Bb_ref[_RB_1A:_RB_1A + 64]))
    Hw = _dotT(Wb_ref[_RW_WV1S:_RW_WV1S + 64, 0:_F], rs(xTw_ref))
    He = _dotT(Wb_ref[_RW_WV1S + 64:_RW_WV1S + 128, 0:_F], rs(xTe_ref))
    Hl = _dotT(Wb_ref[_RW_WV1S + 128:_RW_WV1S + 192, 0:_F], xLb)
    h1w = _elu(_gelu(agg(Aw, dw, Hw[0:32]) + Hw[32:64]
                     + Bb_ref[_RB_1S:_RB_1S + 32]))
    h1e = _elu(_gelu(agg(Ae, de, He[0:32]) + He[32:64]
                     + Bb_ref[_RB_1S + 32:_RB_1S + 64]))
    h1l = _elu(_gelu(agg(Al, dl, Hl[0:32]) + Hl[32:64]
                     + Bb_ref[_RB_1S + 64:_RB_1S + 96]))

    H2a = _mm(Wb_ref[_RW_M2A:_RW_M2A + 48, 0:64], _b16(h1a))
    s_a = jax.nn.relu(agg(Aa, da, H2a[0:24]) + H2a[24:48]
                      + Bb_ref[_RB_2A:_RB_2A + 24])
    H2w = _mm(Wb_ref[_RW_M2S:_RW_M2S + 16, 0:32], _b16(h1w))
    H2e = _mm(Wb_ref[_RW_M2S + 16:_RW_M2S + 32, 0:32], _b16(h1e))
    H2l = _mm(Wb_ref[_RW_M2S + 32:_RW_M2S + 48, 0:32], _b16(h1l))
    s_w = jax.nn.relu(agg(Aw, dw, H2w[0:8]) + H2w[8:16]
                      + Bb_ref[_RB_2S:_RB_2S + 8])
    s_e = jax.nn.relu(agg(Ae, de, H2e[0:8]) + H2e[8:16]
                      + Bb_ref[_RB_2S + 8:_RB_2S + 16])
    s_l = jax.nn.relu(agg(Al, dl, H2l[0:8]) + H2l[8:16]
                      + Bb_ref[_RB_2S + 16:_RB_2S + 24])

    tT = jax.nn.relu(_dotT(Wb_ref[_RW_L1:_RW_L1 + 40, 0:_F], xLb)
                     + Bb_ref[_RB_L1:_RB_L1 + 40])
    s_lt = (_mm(Wb_ref[_RW_L2:_RW_L2 + 8, 0:40], _b16(tT))
            + Bb_ref[_RB_L2:_RB_L2 + 8])

    S = jnp.concatenate([s_a, s_w, s_e, s_l, s_lt], axis=0)
    E = jnp.exp(S - jnp.max(S, axis=0, keepdims=True))
    En = E / _mm(G8_ref[...], _b16(E))
    out_ref[...] = (_mm(Wb_ref[_RW_P8:_RW_P8 + 8, 0:56], _b16(En))
                    + Bb_ref[_RB_F:_RB_F + 8])


def _pad_set(shape, *placements):
    z = jnp.zeros(shape, jnp.float32)
    for (r, c), w in placements:
        z = jax.lax.dynamic_update_slice(z, w, (r, c))
    return z


def kernel(A_input, A_week_input, A_weekend_input, Location_location_input,
           Temporal_input, Temporal_week_input, Temporal_weekend_input,
           Distance_input, Duration_input, Location_time_input,
           W1_temporal, V1_temporal, b1_temporal, W2_temporal, V2_temporal, b2_temporal,
           W1_week, V1_week, b1_week, W2_week, V2_week, b2_week,
           W1_weekend, V1_weekend, b1_weekend, W2_weekend, V2_weekend, b2_weekend,
           W1_distance, V1_distance, b1_distance, W2_distance, V2_distance, b2_distance,
           W1_duration, V1_duration, b1_duration, W2_duration, V2_duration, b2_duration,
           W1_loctime, V1_loctime, b1_loctime, W2_loctime, V2_loctime, b2_loctime,
           L1, bl1, L2, bl2, Ld, bd, Lo, bo):
    f32 = jnp.float32
    Wbuf = _pad_set(
        (_WROWS, 3 * _F),
        ((_RW_M1A + 0, 0), W1_temporal.T),
        ((_RW_M1A + 20, 48), W1_distance.T),
        ((_RW_M1A + 40, 96), W1_duration.T),
        ((_RW_M1A + 64, 0), V1_temporal.T),
        ((_RW_M1A + 84, 48), V1_distance.T),
        ((_RW_M1A + 104, 96), V1_duration.T),
        ((_RW_WV1S + 0, 0), W1_week.T), ((_RW_WV1S + 32, 0), V1_week.T),
        ((_RW_WV1S + 64, 0), W1_weekend.T), ((_RW_WV1S + 96, 0), V1_weekend.T),
        ((_RW_WV1S + 128, 0), W1_loctime.T), ((_RW_WV1S + 160, 0), V1_loctime.T),
        ((_RW_M2A + 0, 0), W2_temporal.T),
        ((_RW_M2A + 8, 20), W2_distance.T),
        ((_RW_M2A + 16, 40), W2_duration.T),
        ((_RW_M2A + 24, 0), V2_temporal.T),
        ((_RW_M2A + 32, 20), V2_distance.T),
        ((_RW_M2A + 40, 40), V2_duration.T),
        ((_RW_M2S + 0, 0), W2_week.T), ((_RW_M2S + 8, 0), V2_week.T),
        ((_RW_M2S + 16, 0), W2_weekend.T), ((_RW_M2S + 24, 0), V2_weekend.T),
        ((_RW_M2S + 32, 0), W2_loctime.T), ((_RW_M2S + 40, 0), V2_loctime.T),
        ((_RW_L1, 0), L1.T),
        ((_RW_L2, 0), L2.T),
        ((_RW_P8, 0), Lo.T), ((_RW_P8, 8), Lo.T), ((_RW_P8, 16), Lo.T),
        ((_RW_P8, 24), Lo.T), ((_RW_P8, 32), Lo.T),
        ((_RW_P8, 40), 2.0 * Ld.T), ((_RW_P8, 48), 2.0 * Ld.T))

    Bbuf = _pad_set(
        (_BROWS, 1),
        ((_RB_1A + 0, 0), b1_temporal[:, None]),
        ((_RB_1A + 20, 0), b1_distance[:, None]),
        ((_RB_1A + 40, 0), b1_duration[:, None]),
        ((_RB_1S + 0, 0), b1_week[:, None]),
        ((_RB_1S + 32, 0), b1_weekend[:, None]),
        ((_RB_1S + 64, 0), b1_loctime[:, None]),
        ((_RB_2A + 0, 0), b2_temporal[:, None]),
        ((_RB_2A + 8, 0), b2_distance[:, None]),
        ((_RB_2A + 16, 0), b2_duration[:, None]),
        ((_RB_2S + 0, 0), b2_week[:, None]),
        ((_RB_2S + 8, 0), b2_weekend[:, None]),
        ((_RB_2S + 16, 0), b2_loctime[:, None]),
        ((_RB_L1, 0), bl1[:, None]),
        ((_RB_L2, 0), bl2[:, None]),
        ((_RB_F, 0), (bd + bo)[:, None]))

    grid = (_B // _GP,)
    badj = pl.BlockSpec((_GP, _M, _M), lambda b: (b, 0, 0))
    bx = pl.BlockSpec((_GP, _M, _F), lambda b: (b, 0, 0))

    def bcast(shape):
        nd = len(shape)
        return pl.BlockSpec(shape, lambda b: (0,) * nd)

    padded = pl.pallas_call(
        _fused_kernel,
        grid=grid,
        in_specs=[badj, badj, badj, badj,
                  bx, bx, bx, bx, bx, bx,
                  bcast((_WROWS, 3 * _F)), bcast((_BROWS, 1)),
                  bcast((56, 56)), bcast((_GP, _GM))],
        out_specs=pl.BlockSpec((8, _GM), lambda b: (0, b)),
        out_shape=jax.ShapeDtypeStruct((8, _N), f32),
        compiler_params=pltpu.CompilerParams(
            dimension_semantics=("parallel",)),
    )(A_input, A_week_input, A_weekend_input, Location_location_input,
      Temporal_input, Temporal_week_input, Temporal_weekend_input,
      Distance_input, Duration_input, Location_time_input,
      _b16(Wbuf), Bbuf,
      jnp.asarray(_G_BLOCK, dtype=jnp.bfloat16),
      jnp.asarray(_SEL, dtype=jnp.bfloat16))
    return padded[:_C].T


# speedup vs baseline: 1.2792x; 1.2792x over previous
import jax
import jax.numpy as jnp
import numpy as np
from jax.experimental import pallas as pl
from jax.experimental.pallas import tpu as pltpu

_B = 64
_M = 128
_F = 48
_C = 7
_H = 20
_N = _B * _M

_GP = 16
_GM = _GP * _M

_SQRT2 = 1.4142135623730951

_REAL = np.array([1.0] * _C + [0.0], dtype=np.float32)
_G_BLOCK = np.kron(np.eye(_C, dtype=np.float32),
                   np.outer(np.ones(8, dtype=np.float32), _REAL))
_SEL = np.kron(np.eye(_GP, dtype=np.float32), np.ones((1, _M), np.float32))

_RW_M1A, _RW_WV1S, _RW_M2A, _RW_M2S = 0, 144, 288, 352
_RW_L1, _RW_L2, _RW_P8, _WROWS = 448, 496, 544, 608
_RB_1A, _RB_1S, _RB_2A, _RB_2S = 0, 64, 160, 184
_RB_L1, _RB_L2, _RB_F, _BROWS = 208, 248, 256, 264


def _gelu(x):
    return 0.5 * x * (1.0 + jax.lax.erf(x / _SQRT2))


def _elu(x):
    return jnp.maximum(x, 0.0) + (jnp.exp(jnp.minimum(x, 0.0)) - 1.0)


def _b16(x):
    return x.astype(jnp.bfloat16)


def _mm(a, b):
    return jnp.dot(a, b, preferred_element_type=jnp.float32)


def _dkm_nk(W, xb):
    return jax.lax.dot_general(W, xb, (((0,), (1,)), ((), ())),
                               preferred_element_type=jnp.float32)


def _dkm_kn(W, hb):
    return jax.lax.dot_general(W, hb, (((0,), (0,)), ((), ())),
                               preferred_element_type=jnp.float32)


def _pack_kernel(W1t, V1t, W2t, V2t, W1w, V1w, W2w, V2w,
                 W1e, V1e, W2e, V2e, W1d, V1d, W2d, V2d,
                 W1u, V1u, W2u, V2u, W1l, V1l, W2l, V2l,
                 L1, L2, Ld, Lo, out_ref):
    out_ref[...] = jnp.zeros((_WROWS, _M), jnp.bfloat16)

    def put(r, c, ref, scale=None):
        v = ref[...]
        if scale is not None:
            v = v * scale
        h, w = v.shape
        out_ref[r:r + h, c:c + w] = _b16(v)

    put(_RW_M1A + 0, 0, W1t)
    put(_RW_M1A + 48, 20, W1d)
    put(_RW_M1A + 96, 40, W1u)
    put(_RW_M1A + 0, 64, V1t)
    put(_RW_M1A + 48, 84, V1d)
    put(_RW_M1A + 96, 104, V1u)
    put(_RW_WV1S + 0, 0, W1w)
    put(_RW_WV1S + 0, 32, V1w)
    put(_RW_WV1S + 48, 0, W1e)
    put(_RW_WV1S + 48, 32, V1e)
    put(_RW_WV1S + 96, 0, W1l)
    put(_RW_WV1S + 96, 32, V1l)
    put(_RW_M2A + 0, 0, W2t)
    put(_RW_M2A + 20, 8, W2d)
    put(_RW_M2A + 40, 16, W2u)
    put(_RW_M2A + 0, 24, V2t)
    put(_RW_M2A + 20, 32, V2d)
    put(_RW_M2A + 40, 40, V2u)
    put(_RW_M2S + 0, 0, W2w)
    put(_RW_M2S + 0, 8, V2w)
    put(_RW_M2S + 32, 0, W2e)
    put(_RW_M2S + 32, 8, V2e)
    put(_RW_M2S + 64, 0, W2l)
    put(_RW_M2S + 64, 8, V2l)
    put(_RW_L1, 0, L1)
    put(_RW_L2, 0, L2)
    for k in range(5):
        put(_RW_P8 + 8 * k, 0, Lo)
    put(_RW_P8 + 40, 0, Ld, scale=2.0)
    put(_RW_P8 + 48, 0, Ld, scale=2.0)


def _fused_kernel(Aa_ref, Aw_ref, Ae_ref, Al_ref,
                  xT_ref, xTw_ref, xTe_ref, xD_ref, xDu_ref, xL_ref,
                  Wb_ref, Bb_ref, G8_ref, Sel_ref,
                  out_ref):
    sel = Sel_ref[...]

    def prep(Aref):
        Ab = _b16(Aref[...]).reshape(_GM, _M)
        deg = _mm(sel, Ab)
        safe = jnp.where(deg > 0, deg, 1.0)
        d = jnp.where(deg > 0, jax.lax.rsqrt(safe), 0.0)
        return Ab, d

    Aa, da = prep(Aa_ref)
    Aw, dw = prep(Aw_ref)
    Ae, de = prep(Ae_ref)
    Al, dl = prep(Al_ref)

    def agg(Ab, d, hT):
        parts = []
        for i in range(_GP):
            di = d[i:i + 1]
            p = _mm(_b16(hT[:, i * _M:(i + 1) * _M] * di),
                    Ab[i * _M:(i + 1) * _M])
            parts.append(p * di)
        return jnp.concatenate(parts, axis=1)

    def rs(ref):
        return _b16(ref[...]).reshape(_GM, _F)

    xa = jnp.concatenate([rs(xT_ref), rs(xD_ref), rs(xDu_ref)], axis=1)
    xLb = rs(xL_ref)
    H1a = _dkm_nk(Wb_ref[_RW_M1A:_RW_M1A + 144], xa)
    h1a = _elu(_gelu(agg(Aa, da, H1a[0:64]) + H1a[64:128]
                     + Bb_ref[_RB_1A:_RB_1A + 64]))
    Hw = _dkm_nk(Wb_ref[_RW_WV1S:_RW_WV1S + 48, 0:64], rs(xTw_ref))
    He = _dkm_nk(Wb_ref[_RW_WV1S + 48:_RW_WV1S + 96, 0:64], rs(xTe_ref))
    Hl = _dkm_nk(Wb_ref[_RW_WV1S + 96:_RW_WV1S + 144, 0:64], xLb)
    h1w = _elu(_gelu(agg(Aw, dw, Hw[0:32]) + Hw[32:64]
                     + Bb_ref[_RB_1S:_RB_1S + 32]))
    h1e = _elu(_gelu(agg(Ae, de, He[0:32]) + He[32:64]
                     + Bb_ref[_RB_1S + 32:_RB_1S + 64]))
    h1l = _elu(_gelu(agg(Al, dl, Hl[0:32]) + Hl[32:64]
                     + Bb_ref[_RB_1S + 64:_RB_1S + 96]))

    H2a = _dkm_kn(Wb_ref[_RW_M2A:_RW_M2A + 64, 0:48], _b16(h1a))
    s_a = jax.nn.relu(agg(Aa, da, H2a[0:24]) + H2a[24:48]
                      + Bb_ref[_RB_2A:_RB_2A + 24])
    H2w = _dkm_kn(Wb_ref[_RW_M2S:_RW_M2S + 32, 0:16], _b16(h1w))
    H2e = _dkm_kn(Wb_ref[_RW_M2S + 32:_RW_M2S + 64, 0:16], _b16(h1e))
    H2l = _dkm_kn(Wb_ref[_RW_M2S + 64:_RW_M2S + 96, 0:16], _b16(h1l))
    s_w = jax.nn.relu(agg(Aw, dw, H2w[0:8]) + H2w[8:16]
                      + Bb_ref[_RB_2S:_RB_2S + 8])
    s_e = jax.nn.relu(agg(Ae, de, H2e[0:8]) + H2e[8:16]
                      + Bb_ref[_RB_2S + 8:_RB_2S + 16])
    s_l = jax.nn.relu(agg(Al, dl, H2l[0:8]) + H2l[8:16]
                      + Bb_ref[_RB_2S + 16:_RB_2S + 24])

    tT = jax.nn.relu(_dkm_nk(Wb_ref[_RW_L1:_RW_L1 + 48, 0:40], xLb)
                     + Bb_ref[_RB_L1:_RB_L1 + 40])
    s_lt = (_dkm_kn(Wb_ref[_RW_L2:_RW_L2 + 40, 0:8], _b16(tT))
            + Bb_ref[_RB_L2:_RB_L2 + 8])

    S = jnp.concatenate([s_a, s_w, s_e, s_l, s_lt], axis=0)
    E = jnp.exp(S - jnp.max(S, axis=0, keepdims=True))
    En = E / _mm(G8_ref[...], _b16(E))
    out_ref[...] = (_dkm_kn(Wb_ref[_RW_P8:_RW_P8 + 56, 0:8], _b16(En))
                    + Bb_ref[_RB_F:_RB_F + 8])


def kernel(A_input, A_week_input, A_weekend_input, Location_location_input,
           Temporal_input, Temporal_week_input, Temporal_weekend_input,
           Distance_input, Duration_input, Location_time_input,
           W1_temporal, V1_temporal, b1_temporal, W2_temporal, V2_temporal, b2_temporal,
           W1_week, V1_week, b1_week, W2_week, V2_week, b2_week,
           W1_weekend, V1_weekend, b1_weekend, W2_weekend, V2_weekend, b2_weekend,
           W1_distance, V1_distance, b1_distance, W2_distance, V2_distance, b2_distance,
           W1_duration, V1_duration, b1_duration, W2_duration, V2_duration, b2_duration,
           W1_loctime, V1_loctime, b1_loctime, W2_loctime, V2_loctime, b2_loctime,
           L1, bl1, L2, bl2, Ld, bd, Lo, bo):
    f32 = jnp.float32

    def z(n):
        return jnp.zeros((n,), f32)

    def sec(b, n):
        return [b, z(n - b.shape[0])]

    bvec = jnp.concatenate(
        [b1_temporal, b1_distance, b1_duration, z(4)]
        + sec(b1_week, 32) + sec(b1_weekend, 32) + sec(b1_loctime, 32)
        + sec(b2_temporal, 8) + sec(b2_distance, 8) + sec(b2_duration, 8)
        + sec(b2_week, 8) + sec(b2_weekend, 8) + sec(b2_loctime, 8)
        + sec(bl1, 40) + sec(bl2, 8) + sec(bd + bo, 8))
    Bbuf = bvec[:, None]

    Wbuf = pl.pallas_call(
        _pack_kernel,
        out_shape=jax.ShapeDtypeStruct((_WROWS, _M), jnp.bfloat16),
    )(W1_temporal, V1_temporal, W2_temporal, V2_temporal,
      W1_week, V1_week, W2_week, V2_week,
      W1_weekend, V1_weekend, W2_weekend, V2_weekend,
      W1_distance, V1_distance, W2_distance, V2_distance,
      W1_duration, V1_duration, W2_duration, V2_duration,
      W1_loctime, V1_loctime, W2_loctime, V2_loctime,
      L1, L2, Ld, Lo)

    grid = (_B // _GP,)
    badj = pl.BlockSpec((_GP, _M, _M), lambda b: (b, 0, 0))
    bx = pl.BlockSpec((_GP, _M, _F), lambda b: (b, 0, 0))

    def bcast(shape):
        nd = len(shape)
        return pl.BlockSpec(shape, lambda b: (0,) * nd)

    padded = pl.pallas_call(
        _fused_kernel,
        grid=grid,
        in_specs=[badj, badj, badj, badj,
                  bx, bx, bx, bx, bx, bx,
                  bcast((_WROWS, _M)), bcast((_BROWS, 1)),
                  bcast((56, 56)), bcast((_GP, _GM))],
        out_specs=pl.BlockSpec((8, _GM), lambda b: (0, b)),
        out_shape=jax.ShapeDtypeStruct((8, _N), f32),
        compiler_params=pltpu.CompilerParams(
            dimension_semantics=("parallel",)),
    )(A_input, A_week_input, A_weekend_input, Location_location_input,
      Temporal_input, Temporal_week_input, Temporal_weekend_input,
      Distance_input, Duration_input, Location_time_input,
      Wbuf, Bbuf,
      jnp.asarray(_G_BLOCK, dtype=jnp.bfloat16),
      jnp.asarray(_SEL, dtype=jnp.bfloat16))
    return padded[:_C].T
